# Initial kernel scaffold; baseline (speedup 1.0000x reference)
import jax
import jax.numpy as jnp
from jax.experimental import pallas as pl


def kernel(x, edge_index, edge_type, edge_emb_table, W_fc, W_fce, attn_l, attn_r, attn_e):
    raise NotImplementedError("write your pallas kernel here")


# speedup vs baseline: 41.6404x; 41.6404x over previous
import functools

import jax
import jax.numpy as jnp
from jax import lax
from jax.experimental import pallas as pl
from jax.experimental.pallas import tpu as pltpu
from jax.experimental.pallas import tpu_sc as plsc

N = 10000
E = 160000
D_IN = 128
H = 2
D_OUT = 64
D_EDGE = 128
NUM_ETYPES = 8
NEG_SLOPE = 0.2

NC = 2
NS = 16
NW = NC * NS
CH = 128
EPAD = 163840
CPT = EPAD // (NW * CH)
EPT = EPAD // NW
DW = 144
ROWS_PT = N // NS
_DUMP = (128, 128, 128, 128, 113)

_BLK = 1000


def _prep_body(x_ref, wfc_ref, alr_ref, eet_ref, wfce_ref, ae_ref,
               hext_ref, nt_ref, ee_ref):
    h = jnp.dot(x_ref[...], wfc_ref[...], preferred_element_type=jnp.float32)
    nt = jnp.dot(h, alr_ref[...], preferred_element_type=jnp.float32)
    nt_ref[...] = nt
    hext_ref[:, 0:D_IN] = h
    li = lax.broadcasted_iota(jnp.int32, (_BLK, DW - D_IN), 1)
    pad = jnp.where(li < H, 1.0, 0.0)
    pad = jnp.where(li == 2, nt[:, 0:1], pad)
    pad = jnp.where(li == 3, nt[:, 1:2], pad)
    hext_ref[:, D_IN:DW] = pad

    @pl.when(pl.program_id(0) == 0)
    def _():
        ef = jnp.dot(eet_ref[...], wfce_ref[...],
                     preferred_element_type=jnp.float32)
        s = ef * ae_ref[...]
        e0 = jnp.sum(s[:, 0:D_EDGE], axis=1, keepdims=True)
        e1 = jnp.sum(s[:, D_EDGE:2 * D_EDGE], axis=1, keepdims=True)
        ee_ref[...] = jnp.concatenate([e0, e1], axis=1)


_prep = pl.pallas_call(
    _prep_body,
    grid=(N // _BLK,),
    in_specs=[
        pl.BlockSpec((_BLK, D_IN), lambda i: (i, 0)),
        pl.BlockSpec((D_IN, D_IN), lambda i: (0, 0)),
        pl.BlockSpec((D_IN, 8), lambda i: (0, 0)),
        pl.BlockSpec((NUM_ETYPES, D_EDGE), lambda i: (0, 0)),
        pl.BlockSpec((D_EDGE, H * D_EDGE), lambda i: (0, 0)),
        pl.BlockSpec((1, H * D_EDGE), lambda i: (0, 0)),
    ],
    out_specs=[
        pl.BlockSpec((_BLK, DW), lambda i: (i, 0)),
        pl.BlockSpec((_BLK, 8), lambda i: (i, 0)),
        pl.BlockSpec((NUM_ETYPES, H), lambda i: (0, 0)),
    ],
    out_shape=[
        jax.ShapeDtypeStruct((N, DW), jnp.float32),
        jax.ShapeDtypeStruct((N, 8), jnp.float32),
        jax.ShapeDtypeStruct((NUM_ETYPES, H), jnp.float32),
    ],
)


def _lrelu(v):
    return jnp.where(v > 0, v, NEG_SLOPE * v)


def _sc_body(hext_hbm, artab_hbm, ee_hbm, row_hbm, col_hbm, et_hbm,
             out_hbm, artab_v, ee_v, rowj_v, colj_v, etj_v, rows_v, w_v,
             acc_sh, sem):
    cid = lax.axis_index("c")
    sid = lax.axis_index("s")
    wid = sid * NC + cid

    pltpu.sync_copy(artab_hbm, artab_v)
    pltpu.sync_copy(ee_hbm, ee_v)

    zeros16 = jnp.zeros((16,), jnp.float32)
    lane = lax.iota(jnp.int32, 16)
    c130 = jnp.full((16,), 130, jnp.int32)
    c131 = jnp.full((16,), 131, jnp.int32)

    def _zrow(r, carry):
        for j in range(DW // 16):
            rows_v[r, pl.ds(j * 16, 16)] = zeros16
        return carry
    lax.fori_loop(0, CH, _zrow, 0)
    off = 0
    for nrows in _DUMP:
        pltpu.sync_copy(rows_v.at[pl.ds(0, nrows)],
                        acc_sh.at[pl.ds(sid * ROWS_PT + off, nrows)])
        off += nrows
    plsc.subcore_barrier()

    def _chunk(j, carry):
        pltpu.sync_copy(row_hbm.at[pl.ds(wid * CPT + j, 1)], rowj_v)
        pltpu.sync_copy(col_hbm.at[pl.ds(wid * CPT + j, 1)], colj_v)
        pltpu.sync_copy(et_hbm.at[pl.ds(wid * CPT + j, 1)], etj_v)
        pltpu.async_copy(hext_hbm.at[rowj_v.at[0]], rows_v, sem).wait()
        ebase = wid * EPT + j * CH

        for g in range(CH // 16):
            sl = pl.ds(g * 16, 16)
            c16 = colj_v[0, sl]
            t16 = etj_v[0, sl]
            rl = g * 16 + lane
            al0 = plsc.load_gather(rows_v, [rl, c130])
            al1 = plsc.load_gather(rows_v, [rl, c131])
            ar0 = plsc.load_gather(artab_v, [c16 * 2])
            ar1 = plsc.load_gather(artab_v, [c16 * 2 + 1])
            ee0 = plsc.load_gather(ee_v, [t16 * 2])
            ee1 = plsc.load_gather(ee_v, [t16 * 2 + 1])
            valid = (ebase + g * 16 + lane) < E
            w0 = jnp.exp(_lrelu(al0 + ar0 + ee0) - _lrelu(ar0))
            w1 = jnp.exp(_lrelu(al1 + ar1 + ee1) - _lrelu(ar1))
            w_v[0, sl] = jnp.where(valid, w0, 0.0)
            w_v[1, sl] = jnp.where(valid, w1, 0.0)

        def _scale(r, carry2):
            w0 = w_v[0, pl.ds(r, 16)][0]
            w1 = w_v[1, pl.ds(r, 16)][0]
            for jj in range(4):
                s2 = pl.ds(jj * 16, 16)
                rows_v[r, s2] = rows_v[r, s2] * w0
            for jj in range(4, 8):
                s2 = pl.ds(jj * 16, 16)
                rows_v[r, s2] = rows_v[r, s2] * w1
            wpad = jnp.where(lane == 0, w0, jnp.where(lane == 1, w1, 0.0))
            rows_v[r, pl.ds(D_IN, 16)] = wpad
            return carry2
        lax.fori_loop(0, CH, _scale, 0)

        pltpu.sync_copy(rows_v, acc_sh.at[colj_v.at[0]], add=True)
        return carry
    lax.fori_loop(0, CPT, _chunk, 0)

    plsc.subcore_barrier()
    off = 0
    for nrows in _DUMP:
        sl = pl.ds(sid * ROWS_PT + off, nrows)
        pltpu.sync_copy(acc_sh.at[sl], rows_v.at[pl.ds(0, nrows)])
        pltpu.sync_copy(rows_v.at[pl.ds(0, nrows)], out_hbm.at[cid].at[sl])
        off += nrows


@functools.cache
def _sc_edge():
    return functools.partial(
        pl.kernel,
        out_type=jax.ShapeDtypeStruct((NC, N, DW), jnp.float32),
        mesh=plsc.VectorSubcoreMesh(core_axis_name="c", subcore_axis_name="s",
                                    num_cores=NC, num_subcores=NS),
        compiler_params=pltpu.CompilerParams(
            needs_layout_passes=False, use_tc_tiling_on_sc=False),
        scratch_types=[
            pltpu.VMEM((2 * N,), jnp.float32),
            pltpu.VMEM((2 * NUM_ETYPES,), jnp.float32),
            pltpu.VMEM((1, CH), jnp.int32),
            pltpu.VMEM((1, CH), jnp.int32),
            pltpu.VMEM((1, CH), jnp.int32),
            pltpu.VMEM((CH, DW), jnp.float32),
            pltpu.VMEM((2, CH + 16), jnp.float32),
            pltpu.VMEM_SHARED((N, DW), jnp.float32),
            pltpu.SemaphoreType.DMA,
        ],
    )(_sc_body)


def _merge_body(acc_ref, out_ref):
    a = acc_ref[0] + acc_ref[1]
    num = a[:, 0:D_IN]
    d0 = a[:, D_IN:D_IN + 1]
    d1 = a[:, D_IN + 1:D_IN + 2]
    li = lax.broadcasted_iota(jnp.int32, (_BLK, D_IN), 1)
    den = jnp.where(li < D_OUT, d0, d1)
    y = num / (den + 1e-16)
    out_ref[...] = jnp.where(y > 0, y, jnp.exp(y) - 1.0)


_merge = pl.pallas_call(
    _merge_body,
    grid=(N // _BLK,),
    in_specs=[pl.BlockSpec((NC, _BLK, DW), lambda i: (0, i, 0))],
    out_specs=pl.BlockSpec((_BLK, D_IN), lambda i: (i, 0)),
    out_shape=jax.ShapeDtypeStruct((N, D_IN), jnp.float32),
)


def kernel(x, edge_index, edge_type, edge_emb_table, W_fc, W_fce,
           attn_l, attn_r, attn_e):
    alr = jnp.zeros((D_IN, 8), jnp.float32)
    alr = alr.at[0:D_OUT, 0].set(attn_l[0, 0])
    alr = alr.at[D_OUT:D_IN, 1].set(attn_l[0, 1])
    alr = alr.at[0:D_OUT, 2].set(attn_r[0, 0])
    alr = alr.at[D_OUT:D_IN, 3].set(attn_r[0, 1])
    ae = attn_e.reshape(1, H * D_EDGE)

    hext, nt8, ee = _prep(x, W_fc, alr, edge_emb_table, W_fce, ae)

    artab = nt8[:, 2:4].reshape(-1)
    ee_flat = ee.reshape(-1)
    pad = jnp.zeros((EPAD - E,), jnp.int32)
    rowp = jnp.concatenate([edge_index[0], pad]).reshape(-1, CH)
    colp = jnp.concatenate([edge_index[1], pad]).reshape(-1, CH)
    etp = jnp.concatenate([edge_type, pad]).reshape(-1, CH)

    acc = _sc_edge()(hext, artab, ee_flat, rowp, colp, etp)
    out = _merge(acc)
    return out.reshape(N, H, D_OUT)
